# baseline (speedup 1.0000x reference)
import functools

import jax
import jax.numpy as jnp
from jax import lax
from jax.experimental import pallas as pl
from jax.experimental.pallas import tpu as pltpu
from jax.experimental.pallas import tpu_sc as plsc

N = 10000
D = 128
G = 64
E = 320000

NC = 2
NS = 16
NW = NC * NS

NP = 10240
RPT = NP // NS
NB = 80
CH = 8
EP = NW * NB * 128
EB = EP // 128
BN = 512
NG = NP // BN

_mesh = plsc.VectorSubcoreMesh(
    core_axis_name="c", subcore_axis_name="s", num_cores=NC, num_subcores=NS
)


@functools.partial(
    pl.kernel,
    out_type=jax.ShapeDtypeStruct((NC * NP, D), jnp.float32),
    mesh=_mesh,
    scratch_types=[
        pltpu.VMEM((NB, 128), jnp.int32),
        pltpu.VMEM((128, D), jnp.float32),
        pltpu.VMEM_SHARED((NP, D), jnp.float32),
    ],
)
def _sc_deg(dst_hbm, ones_hbm, zeros_hbm, out_hbm, dst_v, ones_v, acc_sh):
    cid = lax.axis_index("c")
    sid = lax.axis_index("s")
    wid = cid * NS + sid
    pltpu.sync_copy(dst_hbm.at[pl.ds(wid * NB, NB)], dst_v)
    pltpu.sync_copy(ones_hbm, ones_v)
    pltpu.sync_copy(zeros_hbm, acc_sh.at[pl.ds(sid * RPT, RPT)])
    plsc.subcore_barrier()

    def body(j, carry):
        pltpu.sync_copy(ones_v, acc_sh.at[dst_v.at[j]], add=True)
        return carry

    lax.fori_loop(0, NB, body, 0)
    plsc.subcore_barrier()
    pltpu.sync_copy(
        acc_sh.at[pl.ds(sid * RPT, RPT)],
        out_hbm.at[pl.ds(cid * NP + sid * RPT, RPT)],
    )


@functools.partial(
    pl.kernel,
    out_type=jax.ShapeDtypeStruct((NC * NP, D), jnp.float32),
    mesh=_mesh,
    scratch_types=[
        pltpu.VMEM((CH, 128), jnp.int32),
        pltpu.VMEM((CH, 128), jnp.int32),
        pltpu.VMEM((128, D), jnp.float32),
        pltpu.VMEM((128, D), jnp.float32),
        pltpu.SemaphoreType.DMA,
        pltpu.SemaphoreType.DMA,
        pltpu.SemaphoreType.DMA,
        pltpu.SemaphoreType.DMA,
        pltpu.VMEM_SHARED((NP, D), jnp.float32),
    ],
)
def _sc_agg(hs_hbm, src_hbm, dst_hbm, zeros_hbm, out_hbm,
            src_v, dst_v, rows_a, rows_b, sem_ga, sem_gb, sem_sa, sem_sb,
            acc_sh):
    cid = lax.axis_index("c")
    sid = lax.axis_index("s")
    wid = cid * NS + sid
    pltpu.sync_copy(zeros_hbm, acc_sh.at[pl.ds(sid * RPT, RPT)])
    plsc.subcore_barrier()

    def chunk(cc, carry):
        base = wid * NB + cc * CH
        pltpu.sync_copy(src_hbm.at[pl.ds(base, CH)], src_v)
        pltpu.sync_copy(dst_hbm.at[pl.ds(base, CH)], dst_v)
        pltpu.async_copy(hs_hbm.at[src_v.at[0]], rows_a, sem_ga)
        for k in range(0, CH, 2):
            pltpu.make_async_copy(
                hs_hbm.at[src_v.at[k]], rows_a, sem_ga).wait()
            pltpu.async_copy(rows_a, acc_sh.at[dst_v.at[k]], sem_sa, add=True)
            if k > 0:
                pltpu.make_async_copy(
                    rows_b, acc_sh.at[dst_v.at[k - 1]], sem_sb).wait()
            pltpu.async_copy(hs_hbm.at[src_v.at[k + 1]], rows_b, sem_gb)
            pltpu.make_async_copy(
                hs_hbm.at[src_v.at[k + 1]], rows_b, sem_gb).wait()
            pltpu.async_copy(
                rows_b, acc_sh.at[dst_v.at[k + 1]], sem_sb, add=True)
            pltpu.make_async_copy(
                rows_a, acc_sh.at[dst_v.at[k]], sem_sa).wait()
            if k + 2 < CH:
                pltpu.async_copy(hs_hbm.at[src_v.at[k + 2]], rows_a, sem_ga)
        pltpu.make_async_copy(
            rows_b, acc_sh.at[dst_v.at[CH - 1]], sem_sb).wait()
        return carry

    lax.fori_loop(0, NB // CH, chunk, 0)
    plsc.subcore_barrier()
    pltpu.sync_copy(
        acc_sh.at[pl.ds(sid * RPT, RPT)],
        out_hbm.at[pl.ds(cid * NP + sid * RPT, RPT)],
    )


def _dis_block(degp_ref):
    deg = degp_ref[0, :, 0:1] + degp_ref[1, :, 0:1] + 1.0
    return lax.rsqrt(deg)


def _tc_hs1_body(x_ref, w_ref, degp_ref, out_ref):
    h = jnp.dot(x_ref[...], w_ref[...], preferred_element_type=jnp.float32)
    out_ref[...] = h * _dis_block(degp_ref)


def _tc_mid_body(degp_ref, agg_ref, hs1_ref, b1_ref, w2_ref, out_ref):
    dis = _dis_block(degp_ref)
    tot = agg_ref[0] + agg_ref[1] + hs1_ref[...]
    h1o = jnp.maximum(tot * dis + b1_ref[...], 0.0)
    h2 = jnp.dot(h1o, w2_ref[...], preferred_element_type=jnp.float32)
    out_ref[...] = h2 * dis


def _tc_final_body(degp_ref, agg_ref, hs2_ref, b2_ref, bt_ref, wlt_ref,
                   blin_ref, out_ref, acc_ref, cnt_ref):
    i = pl.program_id(0)

    @pl.when(i == 0)
    def _():
        acc_ref[...] = jnp.zeros_like(acc_ref)
        cnt_ref[...] = jnp.zeros_like(cnt_ref)

    dis = _dis_block(degp_ref)
    h2o = (agg_ref[0] + agg_ref[1] + hs2_ref[...]) * dis + b2_ref[...]
    bt = bt_ref[0]
    gids = lax.broadcasted_iota(jnp.int32, (G, BN), 0)
    onehot = (bt == gids).astype(jnp.float32)
    acc_ref[...] += jnp.dot(onehot, h2o, preferred_element_type=jnp.float32,
                            precision=lax.Precision.HIGHEST)
    cnt_ref[...] += jnp.broadcast_to(
        jnp.sum(onehot, axis=1, keepdims=True), (G, D)
    )

    @pl.when(i == NG - 1)
    def _():
        pooled = acc_ref[...] / jnp.maximum(cnt_ref[...], 1.0)
        r = jnp.sum(pooled * wlt_ref[...], axis=1, keepdims=True)
        out_ref[...] = jnp.broadcast_to(r + blin_ref[0, 0], (G, D))


def _tc_hs1(x_pad, w1, degp):
    return pl.pallas_call(
        _tc_hs1_body,
        grid=(NG,),
        in_specs=[
            pl.BlockSpec((BN, D), lambda i: (i, 0)),
            pl.BlockSpec((D, D), lambda i: (0, 0)),
            pl.BlockSpec((NC, BN, D), lambda i: (0, i, 0)),
        ],
        out_specs=pl.BlockSpec((BN, D), lambda i: (i, 0)),
        out_shape=jax.ShapeDtypeStruct((NP, D), jnp.float32),
    )(x_pad, w1, degp)


def _tc_mid(degp, agg1, hs1, b1, w2):
    return pl.pallas_call(
        _tc_mid_body,
        grid=(NG,),
        in_specs=[
            pl.BlockSpec((NC, BN, D), lambda i: (0, i, 0)),
            pl.BlockSpec((NC, BN, D), lambda i: (0, i, 0)),
            pl.BlockSpec((BN, D), lambda i: (i, 0)),
            pl.BlockSpec((1, D), lambda i: (0, 0)),
            pl.BlockSpec((D, D), lambda i: (0, 0)),
        ],
        out_specs=pl.BlockSpec((BN, D), lambda i: (i, 0)),
        out_shape=jax.ShapeDtypeStruct((NP, D), jnp.float32),
    )(degp, agg1, hs1, b1, w2)


def _tc_final(degp, agg2, hs2, b2, bt3d, wlt, blin2):
    return pl.pallas_call(
        _tc_final_body,
        grid=(NG,),
        in_specs=[
            pl.BlockSpec((NC, BN, D), lambda i: (0, i, 0)),
            pl.BlockSpec((NC, BN, D), lambda i: (0, i, 0)),
            pl.BlockSpec((BN, D), lambda i: (i, 0)),
            pl.BlockSpec((1, D), lambda i: (0, 0)),
            pl.BlockSpec((1, 1, BN), lambda i: (i, 0, 0)),
            pl.BlockSpec((1, D), lambda i: (0, 0)),
            pl.BlockSpec((1, 1), lambda i: (0, 0)),
        ],
        out_specs=pl.BlockSpec((G, D), lambda i: (0, 0)),
        out_shape=jax.ShapeDtypeStruct((G, D), jnp.float32),
        scratch_shapes=[
            pltpu.VMEM((G, D), jnp.float32),
            pltpu.VMEM((G, D), jnp.float32),
        ],
    )(degp, agg2, hs2, b2, bt3d, wlt, blin2)


def kernel(x, edge_index, batch, W1, b1, W2, b2, Wlin, blin):
    f32 = jnp.float32
    x_pad = jnp.zeros((NP, D), f32).at[:N].set(x)

    npad = EP - E
    pad_idx = (N + jnp.arange(npad, dtype=jnp.int32) % (NP - N)).astype(
        edge_index.dtype
    )
    src2d = jnp.concatenate([edge_index[0], pad_idx]).reshape(EB, 128)
    dst2d = jnp.concatenate([edge_index[1], pad_idx]).reshape(EB, 128)

    bt3d = jnp.concatenate(
        [batch, jnp.full((NP - N,), G, batch.dtype)]
    ).reshape(NG, 1, BN)

    onesD = jnp.ones((128, D), f32)
    zerosD = jnp.zeros((RPT, D), f32)

    degp = _sc_deg(dst2d, onesD, zerosD).reshape(NC, NP, D)
    hs1 = _tc_hs1(x_pad, W1, degp)
    agg1 = _sc_agg(hs1, src2d, dst2d, zerosD).reshape(NC, NP, D)
    hs2 = _tc_mid(degp, agg1, hs1, b1.reshape(1, D), W2)
    agg2 = _sc_agg(hs2, src2d, dst2d, zerosD).reshape(NC, NP, D)
    outf = _tc_final(
        degp, agg2, hs2, b2.reshape(1, D), bt3d,
        Wlin.reshape(1, D), blin.reshape(1, 1),
    )
    return outf[:, :1]


# speedup vs baseline: 1.0344x; 1.0344x over previous
import functools

import jax
import jax.numpy as jnp
from jax import lax
from jax.experimental import pallas as pl
from jax.experimental.pallas import tpu as pltpu
from jax.experimental.pallas import tpu_sc as plsc

N = 10000
D = 128
G = 64
E = 320000

NC = 2
NS = 16
NW = NC * NS

NP = 10240
RPT = NP // NS
NB = 80
CH = 16
EP = NW * NB * 128
EB = EP // 128
BN = 512
NG = NP // BN

_mesh = plsc.VectorSubcoreMesh(
    core_axis_name="c", subcore_axis_name="s", num_cores=NC, num_subcores=NS
)


@functools.partial(
    pl.kernel,
    out_type=jax.ShapeDtypeStruct((NC * NP, D), jnp.float32),
    mesh=_mesh,
    scratch_types=[
        pltpu.VMEM((NB, 128), jnp.int32),
        pltpu.VMEM((128, D), jnp.float32),
        pltpu.VMEM_SHARED((NP, D), jnp.float32),
    ],
)
def _sc_deg(dst_hbm, ones_hbm, zeros_hbm, out_hbm, dst_v, ones_v, acc_sh):
    cid = lax.axis_index("c")
    sid = lax.axis_index("s")
    wid = cid * NS + sid
    pltpu.sync_copy(dst_hbm.at[pl.ds(wid * NB, NB)], dst_v)
    pltpu.sync_copy(ones_hbm, ones_v)
    pltpu.sync_copy(zeros_hbm, acc_sh.at[pl.ds(sid * RPT, RPT)])
    plsc.subcore_barrier()

    def body(j, carry):
        pltpu.sync_copy(ones_v, acc_sh.at[dst_v.at[j]], add=True)
        return carry

    lax.fori_loop(0, NB, body, 0)
    plsc.subcore_barrier()
    pltpu.sync_copy(
        acc_sh.at[pl.ds(sid * RPT, RPT)],
        out_hbm.at[pl.ds(cid * NP + sid * RPT, RPT)],
    )


@functools.partial(
    pl.kernel,
    out_type=jax.ShapeDtypeStruct((NC * NP, D), jnp.float32),
    mesh=_mesh,
    scratch_types=[
        pltpu.VMEM((CH, 128), jnp.int32),
        pltpu.VMEM((CH, 128), jnp.int32),
        pltpu.VMEM((128, D), jnp.float32),
        pltpu.VMEM((128, D), jnp.float32),
        pltpu.SemaphoreType.DMA,
        pltpu.SemaphoreType.DMA,
        pltpu.VMEM_SHARED((NP, D), jnp.float32),
    ],
)
def _sc_agg(hs_hbm, src_hbm, dst_hbm, zeros_hbm, out_hbm,
            src_v, dst_v, rows_a, rows_b, sem_a, sem_b, acc_sh):
    cid = lax.axis_index("c")
    sid = lax.axis_index("s")
    wid = cid * NS + sid
    pltpu.sync_copy(zeros_hbm, acc_sh.at[pl.ds(sid * RPT, RPT)])
    plsc.subcore_barrier()

    def chunk(cc, carry):
        base = wid * NB + cc * CH
        pltpu.sync_copy(src_hbm.at[pl.ds(base, CH)], src_v)
        pltpu.sync_copy(dst_hbm.at[pl.ds(base, CH)], dst_v)
        pltpu.async_copy(hs_hbm.at[src_v.at[0]], rows_a, sem_a)
        for k in range(0, CH, 2):
            pltpu.make_async_copy(hs_hbm.at[src_v.at[k]], rows_a, sem_a).wait()
            pltpu.async_copy(hs_hbm.at[src_v.at[k + 1]], rows_b, sem_b)
            pltpu.sync_copy(rows_a, acc_sh.at[dst_v.at[k]], add=True)
            pltpu.make_async_copy(
                hs_hbm.at[src_v.at[k + 1]], rows_b, sem_b).wait()
            if k + 2 < CH:
                pltpu.async_copy(hs_hbm.at[src_v.at[k + 2]], rows_a, sem_a)
            pltpu.sync_copy(rows_b, acc_sh.at[dst_v.at[k + 1]], add=True)
        return carry

    lax.fori_loop(0, NB // CH, chunk, 0)
    plsc.subcore_barrier()
    pltpu.sync_copy(
        acc_sh.at[pl.ds(sid * RPT, RPT)],
        out_hbm.at[pl.ds(cid * NP + sid * RPT, RPT)],
    )


def _dis_block(degp_ref):
    deg = degp_ref[0, :, 0:1] + degp_ref[1, :, 0:1] + 1.0
    return lax.rsqrt(deg)


def _tc_hs1_body(x_ref, w_ref, degp_ref, out_ref):
    h = jnp.dot(x_ref[...], w_ref[...], preferred_element_type=jnp.float32)
    out_ref[...] = h * _dis_block(degp_ref)


def _tc_mid_body(degp_ref, agg_ref, hs1_ref, b1_ref, w2_ref, out_ref):
    dis = _dis_block(degp_ref)
    tot = agg_ref[0] + agg_ref[1] + hs1_ref[...]
    h1o = jnp.maximum(tot * dis + b1_ref[...], 0.0)
    h2 = jnp.dot(h1o, w2_ref[...], preferred_element_type=jnp.float32)
    out_ref[...] = h2 * dis


def _tc_final_body(degp_ref, agg_ref, hs2_ref, b2_ref, bt_ref, wlt_ref,
                   blin_ref, out_ref, acc_ref, cnt_ref):
    i = pl.program_id(0)

    @pl.when(i == 0)
    def _():
        acc_ref[...] = jnp.zeros_like(acc_ref)
        cnt_ref[...] = jnp.zeros_like(cnt_ref)

    dis = _dis_block(degp_ref)
    h2o = (agg_ref[0] + agg_ref[1] + hs2_ref[...]) * dis + b2_ref[...]
    bt = bt_ref[0]
    gids = lax.broadcasted_iota(jnp.int32, (G, BN), 0)
    onehot = (bt == gids).astype(jnp.float32)
    acc_ref[...] += jnp.dot(onehot, h2o, preferred_element_type=jnp.float32,
                            precision=lax.Precision.HIGHEST)
    cnt_ref[...] += jnp.broadcast_to(
        jnp.sum(onehot, axis=1, keepdims=True), (G, D)
    )

    @pl.when(i == NG - 1)
    def _():
        pooled = acc_ref[...] / jnp.maximum(cnt_ref[...], 1.0)
        r = jnp.sum(pooled * wlt_ref[...], axis=1, keepdims=True)
        out_ref[...] = jnp.broadcast_to(r + blin_ref[0, 0], (G, D))


def _tc_hs1(x_pad, w1, degp):
    return pl.pallas_call(
        _tc_hs1_body,
        grid=(NG,),
        in_specs=[
            pl.BlockSpec((BN, D), lambda i: (i, 0)),
            pl.BlockSpec((D, D), lambda i: (0, 0)),
            pl.BlockSpec((NC, BN, D), lambda i: (0, i, 0)),
        ],
        out_specs=pl.BlockSpec((BN, D), lambda i: (i, 0)),
        out_shape=jax.ShapeDtypeStruct((NP, D), jnp.float32),
    )(x_pad, w1, degp)


def _tc_mid(degp, agg1, hs1, b1, w2):
    return pl.pallas_call(
        _tc_mid_body,
        grid=(NG,),
        in_specs=[
            pl.BlockSpec((NC, BN, D), lambda i: (0, i, 0)),
            pl.BlockSpec((NC, BN, D), lambda i: (0, i, 0)),
            pl.BlockSpec((BN, D), lambda i: (i, 0)),
            pl.BlockSpec((1, D), lambda i: (0, 0)),
            pl.BlockSpec((D, D), lambda i: (0, 0)),
        ],
        out_specs=pl.BlockSpec((BN, D), lambda i: (i, 0)),
        out_shape=jax.ShapeDtypeStruct((NP, D), jnp.float32),
    )(degp, agg1, hs1, b1, w2)


def _tc_final(degp, agg2, hs2, b2, bt3d, wlt, blin2):
    return pl.pallas_call(
        _tc_final_body,
        grid=(NG,),
        in_specs=[
            pl.BlockSpec((NC, BN, D), lambda i: (0, i, 0)),
            pl.BlockSpec((NC, BN, D), lambda i: (0, i, 0)),
            pl.BlockSpec((BN, D), lambda i: (i, 0)),
            pl.BlockSpec((1, D), lambda i: (0, 0)),
            pl.BlockSpec((1, 1, BN), lambda i: (i, 0, 0)),
            pl.BlockSpec((1, D), lambda i: (0, 0)),
            pl.BlockSpec((1, 1), lambda i: (0, 0)),
        ],
        out_specs=pl.BlockSpec((G, D), lambda i: (0, 0)),
        out_shape=jax.ShapeDtypeStruct((G, D), jnp.float32),
        scratch_shapes=[
            pltpu.VMEM((G, D), jnp.float32),
            pltpu.VMEM((G, D), jnp.float32),
        ],
    )(degp, agg2, hs2, b2, bt3d, wlt, blin2)


def kernel(x, edge_index, batch, W1, b1, W2, b2, Wlin, blin):
    f32 = jnp.float32
    x_pad = jnp.zeros((NP, D), f32).at[:N].set(x)

    npad = EP - E
    pad_idx = (N + jnp.arange(npad, dtype=jnp.int32) % (NP - N)).astype(
        edge_index.dtype
    )
    src2d = jnp.concatenate([edge_index[0], pad_idx]).reshape(EB, 128)
    dst2d = jnp.concatenate([edge_index[1], pad_idx]).reshape(EB, 128)

    bt3d = jnp.concatenate(
        [batch, jnp.full((NP - N,), G, batch.dtype)]
    ).reshape(NG, 1, BN)

    onesD = jnp.ones((128, D), f32)
    zerosD = jnp.zeros((RPT, D), f32)

    degp = _sc_deg(dst2d, onesD, zerosD).reshape(NC, NP, D)
    hs1 = _tc_hs1(x_pad, W1, degp)
    agg1 = _sc_agg(hs1, src2d, dst2d, zerosD).reshape(NC, NP, D)
    hs2 = _tc_mid(degp, agg1, hs1, b1.reshape(1, D), W2)
    agg2 = _sc_agg(hs2, src2d, dst2d, zerosD).reshape(NC, NP, D)
    outf = _tc_final(
        degp, agg2, hs2, b2.reshape(1, D), bt3d,
        Wlin.reshape(1, D), blin.reshape(1, 1),
    )
    return outf[:, :1]


# speedup vs baseline: 1.0398x; 1.0052x over previous
import functools

import jax
import jax.numpy as jnp
from jax import lax
from jax.experimental import pallas as pl
from jax.experimental.pallas import tpu as pltpu
from jax.experimental.pallas import tpu_sc as plsc

N = 10000
D = 128
G = 64
E = 320000

NC = 2
NS = 16
NW = NC * NS

NP = 10240
RPT = NP // NS
NB = 80
CH = 16
EP = NW * NB * 128
EB = EP // 128
BN = 512
NG = NP // BN

_mesh = plsc.VectorSubcoreMesh(
    core_axis_name="c", subcore_axis_name="s", num_cores=NC, num_subcores=NS
)


@functools.partial(
    pl.kernel,
    out_type=jax.ShapeDtypeStruct((NC * NP, D), jnp.float32),
    mesh=_mesh,
    scratch_types=[
        pltpu.VMEM((NB, 128), jnp.int32),
        pltpu.VMEM((128, D), jnp.float32),
        pltpu.SemaphoreType.DMA,
        pltpu.VMEM_SHARED((NP, D), jnp.float32),
    ],
)
def _sc_deg(dst_hbm, ones_hbm, zeros_hbm, out_hbm, dst_v, ones_v, sem,
            acc_sh):
    cid = lax.axis_index("c")
    sid = lax.axis_index("s")
    wid = cid * NS + sid
    pltpu.sync_copy(dst_hbm.at[pl.ds(wid * NB, NB)], dst_v)
    pltpu.sync_copy(ones_hbm, ones_v)
    pltpu.sync_copy(zeros_hbm, acc_sh.at[pl.ds(sid * RPT, RPT)])
    plsc.subcore_barrier()

    def body(j, carry):
        base = j * CH
        for k in range(CH):
            pltpu.async_copy(
                ones_v, acc_sh.at[dst_v.at[base + k]], sem, add=True)
        for k in range(CH):
            pltpu.make_async_copy(
                ones_v, acc_sh.at[dst_v.at[base + k]], sem).wait()
        return carry

    lax.fori_loop(0, NB // CH, body, 0)
    plsc.subcore_barrier()
    pltpu.sync_copy(
        acc_sh.at[pl.ds(sid * RPT, RPT)],
        out_hbm.at[pl.ds(cid * NP + sid * RPT, RPT)],
    )


@functools.partial(
    pl.kernel,
    out_type=jax.ShapeDtypeStruct((NC * NP, D), jnp.float32),
    mesh=_mesh,
    scratch_types=[
        pltpu.VMEM((CH, 128), jnp.int32),
        pltpu.VMEM((CH, 128), jnp.int32),
        pltpu.VMEM((128, D), jnp.float32),
        pltpu.VMEM((128, D), jnp.float32),
        pltpu.SemaphoreType.DMA,
        pltpu.SemaphoreType.DMA,
        pltpu.VMEM_SHARED((NP, D), jnp.float32),
    ],
)
def _sc_agg(hs_hbm, src_hbm, dst_hbm, zeros_hbm, out_hbm,
            src_v, dst_v, rows_a, rows_b, sem_a, sem_b, acc_sh):
    cid = lax.axis_index("c")
    sid = lax.axis_index("s")
    wid = cid * NS + sid
    pltpu.sync_copy(zeros_hbm, acc_sh.at[pl.ds(sid * RPT, RPT)])
    plsc.subcore_barrier()

    def chunk(cc, carry):
        base = wid * NB + cc * CH
        pltpu.sync_copy(src_hbm.at[pl.ds(base, CH)], src_v)
        pltpu.sync_copy(dst_hbm.at[pl.ds(base, CH)], dst_v)
        pltpu.async_copy(hs_hbm.at[src_v.at[0]], rows_a, sem_a)
        for k in range(0, CH, 2):
            pltpu.make_async_copy(hs_hbm.at[src_v.at[k]], rows_a, sem_a).wait()
            pltpu.async_copy(hs_hbm.at[src_v.at[k + 1]], rows_b, sem_b)
            pltpu.sync_copy(rows_a, acc_sh.at[dst_v.at[k]], add=True)
            pltpu.make_async_copy(
                hs_hbm.at[src_v.at[k + 1]], rows_b, sem_b).wait()
            if k + 2 < CH:
                pltpu.async_copy(hs_hbm.at[src_v.at[k + 2]], rows_a, sem_a)
            pltpu.sync_copy(rows_b, acc_sh.at[dst_v.at[k + 1]], add=True)
        return carry

    lax.fori_loop(0, NB // CH, chunk, 0)
    plsc.subcore_barrier()
    pltpu.sync_copy(
        acc_sh.at[pl.ds(sid * RPT, RPT)],
        out_hbm.at[pl.ds(cid * NP + sid * RPT, RPT)],
    )


def _dis_block(degp_ref):
    deg = degp_ref[0, :, 0:1] + degp_ref[1, :, 0:1] + 1.0
    return lax.rsqrt(deg)


def _tc_hs1_body(x_ref, w_ref, degp_ref, out_ref):
    h = jnp.dot(x_ref[...], w_ref[...], preferred_element_type=jnp.float32)
    out_ref[...] = h * _dis_block(degp_ref)


def _tc_mid_body(degp_ref, agg_ref, hs1_ref, b1_ref, w2_ref, out_ref):
    dis = _dis_block(degp_ref)
    tot = agg_ref[0] + agg_ref[1] + hs1_ref[...]
    h1o = jnp.maximum(tot * dis + b1_ref[...], 0.0)
    h2 = jnp.dot(h1o, w2_ref[...], preferred_element_type=jnp.float32)
    out_ref[...] = h2 * dis


def _tc_final_body(degp_ref, agg_ref, hs2_ref, b2_ref, bt_ref, wlt_ref,
                   blin_ref, out_ref, acc_ref, cnt_ref):
    i = pl.program_id(0)

    @pl.when(i == 0)
    def _():
        acc_ref[...] = jnp.zeros_like(acc_ref)
        cnt_ref[...] = jnp.zeros_like(cnt_ref)

    dis = _dis_block(degp_ref)
    h2o = (agg_ref[0] + agg_ref[1] + hs2_ref[...]) * dis + b2_ref[...]
    bt = bt_ref[0]
    gids = lax.broadcasted_iota(jnp.int32, (G, BN), 0)
    onehot = (bt == gids).astype(jnp.float32)
    acc_ref[...] += jnp.dot(onehot, h2o, preferred_element_type=jnp.float32,
                            precision=lax.Precision.HIGHEST)
    cnt_ref[...] += jnp.broadcast_to(
        jnp.sum(onehot, axis=1, keepdims=True), (G, D)
    )

    @pl.when(i == NG - 1)
    def _():
        pooled = acc_ref[...] / jnp.maximum(cnt_ref[...], 1.0)
        r = jnp.sum(pooled * wlt_ref[...], axis=1, keepdims=True)
        out_ref[...] = jnp.broadcast_to(r + blin_ref[0, 0], (G, D))


def _tc_hs1(x_pad, w1, degp):
    return pl.pallas_call(
        _tc_hs1_body,
        grid=(NG,),
        in_specs=[
            pl.BlockSpec((BN, D), lambda i: (i, 0)),
            pl.BlockSpec((D, D), lambda i: (0, 0)),
            pl.BlockSpec((NC, BN, D), lambda i: (0, i, 0)),
        ],
        out_specs=pl.BlockSpec((BN, D), lambda i: (i, 0)),
        out_shape=jax.ShapeDtypeStruct((NP, D), jnp.float32),
    )(x_pad, w1, degp)


def _tc_mid(degp, agg1, hs1, b1, w2):
    return pl.pallas_call(
        _tc_mid_body,
        grid=(NG,),
        in_specs=[
            pl.BlockSpec((NC, BN, D), lambda i: (0, i, 0)),
            pl.BlockSpec((NC, BN, D), lambda i: (0, i, 0)),
            pl.BlockSpec((BN, D), lambda i: (i, 0)),
            pl.BlockSpec((1, D), lambda i: (0, 0)),
            pl.BlockSpec((D, D), lambda i: (0, 0)),
        ],
        out_specs=pl.BlockSpec((BN, D), lambda i: (i, 0)),
        out_shape=jax.ShapeDtypeStruct((NP, D), jnp.float32),
    )(degp, agg1, hs1, b1, w2)


def _tc_final(degp, agg2, hs2, b2, bt3d, wlt, blin2):
    return pl.pallas_call(
        _tc_final_body,
        grid=(NG,),
        in_specs=[
            pl.BlockSpec((NC, BN, D), lambda i: (0, i, 0)),
            pl.BlockSpec((NC, BN, D), lambda i: (0, i, 0)),
            pl.BlockSpec((BN, D), lambda i: (i, 0)),
            pl.BlockSpec((1, D), lambda i: (0, 0)),
            pl.BlockSpec((1, 1, BN), lambda i: (i, 0, 0)),
            pl.BlockSpec((1, D), lambda i: (0, 0)),
            pl.BlockSpec((1, 1), lambda i: (0, 0)),
        ],
        out_specs=pl.BlockSpec((G, D), lambda i: (0, 0)),
        out_shape=jax.ShapeDtypeStruct((G, D), jnp.float32),
        scratch_shapes=[
            pltpu.VMEM((G, D), jnp.float32),
            pltpu.VMEM((G, D), jnp.float32),
        ],
    )(degp, agg2, hs2, b2, bt3d, wlt, blin2)


def kernel(x, edge_index, batch, W1, b1, W2, b2, Wlin, blin):
    f32 = jnp.float32
    x_pad = jnp.zeros((NP, D), f32).at[:N].set(x)

    npad = EP - E
    pad_idx = (N + jnp.arange(npad, dtype=jnp.int32) % (NP - N)).astype(
        edge_index.dtype
    )
    src2d = jnp.concatenate([edge_index[0], pad_idx]).reshape(EB, 128)
    dst2d = jnp.concatenate([edge_index[1], pad_idx]).reshape(EB, 128)

    bt3d = jnp.concatenate(
        [batch, jnp.full((NP - N,), G, batch.dtype)]
    ).reshape(NG, 1, BN)

    onesD = jnp.ones((128, D), f32)
    zerosD = jnp.zeros((RPT, D), f32)

    degp = _sc_deg(dst2d, onesD, zerosD).reshape(NC, NP, D)
    hs1 = _tc_hs1(x_pad, W1, degp)
    agg1 = _sc_agg(hs1, src2d, dst2d, zerosD).reshape(NC, NP, D)
    hs2 = _tc_mid(degp, agg1, hs1, b1.reshape(1, D), W2)
    agg2 = _sc_agg(hs2, src2d, dst2d, zerosD).reshape(NC, NP, D)
    outf = _tc_final(
        degp, agg2, hs2, b2.reshape(1, D), bt3d,
        Wlin.reshape(1, D), blin.reshape(1, 1),
    )
    return outf[:, :1]
